# Initial kernel scaffold; baseline (speedup 1.0000x reference)
import jax
import jax.numpy as jnp
from jax.experimental import pallas as pl


def kernel(x, router_w, router_b, fc1_w, fc1_b, fc2_w, fc2_b):
    raise NotImplementedError("write your pallas kernel here")


# speedup vs baseline: 2.0350x; 2.0350x over previous
import functools

import jax
import jax.numpy as jnp
from jax.experimental import pallas as pl
from jax.experimental.pallas import tpu as pltpu

_NEG_INF = float("-inf")


def _moe_dense_kernel(x_ref, rw_ref, rb_ref, fc1w_ref, fc1b_ref,
                      fc2w_ref, fc2b_ref, out_ref, *, tile_t, n_experts):
    e = pl.program_id(0)
    t = pl.program_id(1)

    x_t = x_ref[0, pl.ds(t * tile_t, tile_t), :]

    logits = jax.lax.dot_general(
        x_t, rw_ref[...],
        dimension_numbers=(((1,), (1,)), ((), ())),
        preferred_element_type=jnp.float32,
    ) + rb_ref[...][None, :]

    col = jax.lax.broadcasted_iota(jnp.int32, logits.shape, 1)
    m1 = jnp.max(logits, axis=1, keepdims=True)
    i1 = jnp.min(jnp.where(logits == m1, col, n_experts), axis=1)
    l2 = jnp.where(col == i1[:, None], _NEG_INF, logits)
    m2 = jnp.max(l2, axis=1, keepdims=True)
    i2 = jnp.min(jnp.where(l2 == m2, col, n_experts), axis=1)
    b = jnp.exp(m2[:, 0] - m1[:, 0])
    w1 = 1.0 / (1.0 + b)
    w2 = 1.0 - w1
    cw = w1 * (i1 == e).astype(jnp.float32) + w2 * (i2 == e).astype(jnp.float32)

    h = jax.lax.dot_general(
        x_t, fc1w_ref[0],
        dimension_numbers=(((1,), (1,)), ((), ())),
        preferred_element_type=jnp.float32,
    ) + fc1b_ref[pl.ds(e, 1), :]
    hdim = h.shape[1] // 2
    h1 = h[:, :hdim]
    h2 = h[:, hdim:]
    g = h1 * jax.nn.sigmoid(h1) * h2
    y = jax.lax.dot_general(
        g, fc2w_ref[0],
        dimension_numbers=(((1,), (1,)), ((), ())),
        preferred_element_type=jnp.float32,
    ) + fc2b_ref[pl.ds(e, 1), :]

    contrib = cw[:, None] * y

    @pl.when(e == 0)
    def _init():
        out_ref[0, pl.ds(t * tile_t, tile_t), :] = contrib

    @pl.when(e != 0)
    def _acc():
        out_ref[0, pl.ds(t * tile_t, tile_t), :] += contrib


def kernel(x, router_w, router_b, fc1_w, fc1_b, fc2_w, fc2_b):
    B, T, D = x.shape
    E, H2, _ = fc1_w.shape
    tile_t = 512
    n_t = T // tile_t

    grid = (E, n_t)
    return pl.pallas_call(
        functools.partial(_moe_dense_kernel, tile_t=tile_t, n_experts=E),
        grid=grid,
        in_specs=[
            pl.BlockSpec((B, T, D), lambda e, t: (0, 0, 0)),
            pl.BlockSpec((E, D), lambda e, t: (0, 0)),
            pl.BlockSpec((E,), lambda e, t: (0,)),
            pl.BlockSpec((1, H2, D), lambda e, t: (e, 0, 0)),
            pl.BlockSpec((E, H2), lambda e, t: (0, 0)),
            pl.BlockSpec((1, D, H2 // 2), lambda e, t: (e, 0, 0)),
            pl.BlockSpec((E, D), lambda e, t: (0, 0)),
        ],
        out_specs=pl.BlockSpec((B, T, D), lambda e, t: (0, 0, 0)),
        out_shape=jax.ShapeDtypeStruct((B, T, D), x.dtype),
        compiler_params=pltpu.CompilerParams(
            dimension_semantics=("arbitrary", "arbitrary"),
        ),
    )(x, router_w, router_b, fc1_w, fc1_b, fc2_w, fc2_b)
